# baseline (speedup 1.0000x reference)
import functools

import jax
import jax.numpy as jnp
from jax import lax
from jax.experimental import pallas as pl
from jax.experimental.pallas import tpu as pltpu
from jax.experimental.pallas import tpu_sc as plsc

N_USERS = 1_000_000
N_ITEMS = 1_000_000
D = 16
BATCH = 16384
HIST = 50

NC = 2
NS = 16
NW = NC * NS

USERS_PW = BATCH // NW
BLK_U = 64
BLK_R = BLK_U * HIST
N_BLK = USERS_PW // BLK_U
GROUPS = BLK_R // D
CHUNK = 128
N_CHUNK = BLK_R // CHUNK


def _mf_body(users_hbm, items_hbm, utab_hbm, itab_hbm, out_hbm,
             idx_v, uidx_v, u_v, it_v, out_v, sem):
    wid = lax.axis_index("s") * NC + lax.axis_index("c")
    iota16 = lax.iota(jnp.int32, 16)

    def block(b, carry):
        row_base = wid * (USERS_PW * HIST) + b * BLK_R
        u_base = wid * USERS_PW + b * BLK_U

        pltpu.sync_copy(items_hbm.at[pl.ds(row_base, BLK_R)], idx_v)
        pltpu.sync_copy(users_hbm.at[pl.ds(u_base, BLK_U)], uidx_v)
        pltpu.async_copy(utab_hbm.at[uidx_v], u_v, sem).wait()

        copies = []
        for j in range(N_CHUNK):
            copies.append(pltpu.async_copy(
                itab_hbm.at[idx_v.at[pl.ds(j * CHUNK, CHUNK)]],
                it_v.at[pl.ds(j * CHUNK, CHUNK)],
                sem))
        for c in copies:
            c.wait()

        def group(g, carry2):
            rows = g * 16 + iota16
            urows = rows // HIST
            acc = jnp.zeros((16,), jnp.float32)
            for d in range(D):
                dcol = jnp.full((16,), d, jnp.int32)
                ic = plsc.load_gather(it_v, [rows, dcol])
                uc = plsc.load_gather(u_v, [urows, dcol])
                acc = acc + ic * uc
            out_v[pl.ds(g * 16, 16)] = acc
            return carry2

        lax.fori_loop(0, GROUPS, group, 0, unroll=False)
        pltpu.sync_copy(out_v, out_hbm.at[pl.ds(row_base, BLK_R)])
        return carry

    lax.fori_loop(0, N_BLK, block, 0, unroll=False)


@jax.jit
def _mf(users, items_flat, user_table, item_table):
    mesh = plsc.VectorSubcoreMesh(core_axis_name="c", subcore_axis_name="s",
                                  num_cores=NC, num_subcores=NS)
    return pl.kernel(
        _mf_body,
        out_type=jax.ShapeDtypeStruct((BATCH * HIST,), jnp.float32),
        mesh=mesh,
        scratch_types=[
            pltpu.VMEM((BLK_R,), jnp.int32),
            pltpu.VMEM((BLK_U,), jnp.int32),
            pltpu.VMEM((BLK_U, D), jnp.float32),
            pltpu.VMEM((BLK_R, D), jnp.float32),
            pltpu.VMEM((BLK_R,), jnp.float32),
            pltpu.SemaphoreType.DMA,
        ],
        compiler_params=pltpu.CompilerParams(
            needs_layout_passes=False, use_tc_tiling_on_sc=False),
    )(users, items_flat, user_table, item_table)


def kernel(users, items, user_table, item_table):
    users = users.astype(jnp.int32)
    items_flat = items.astype(jnp.int32).reshape(-1)
    out_flat = _mf(users, items_flat, user_table, item_table)
    return out_flat.reshape(BATCH, HIST)
